# Initial kernel scaffold; baseline (speedup 1.0000x reference)
import jax
import jax.numpy as jnp
from jax.experimental import pallas as pl


def kernel(x_og, x_cg, pos_og, pos_cg, og_to_cg_edge_index, og_to_cg_edge_attr, edge_index_cg, x_og_batch, x_cg_batch, params):
    raise NotImplementedError("write your pallas kernel here")


# speedup vs baseline: 1.2768x; 1.2768x over previous
import functools

import jax
import jax.numpy as jnp
from jax import lax
from jax.experimental import pallas as pl
from jax.experimental.pallas import tpu as pltpu
from jax.experimental.pallas import tpu_sc as plsc

F32 = jnp.float32
BF16 = jnp.bfloat16
H = 128
EPS = 1e-5
NW = 32
CH = 128
BE = 2000


def _sds(shape, dtype=F32):
    return jax.ShapeDtypeStruct(shape, dtype)


def _mm(a, b):
    return jnp.dot(a.astype(BF16), b.astype(BF16), preferred_element_type=F32)


def _mm2(a2, w2):
    a = a2.astype(BF16).astype(F32)
    w = w2.astype(BF16).astype(F32)
    return a[:, 0:1] * w[0:1, :] + a[:, 1:2] * w[1:2, :]


def _bn_relu(x):
    m = jnp.mean(x, axis=0, keepdims=True)
    v = jnp.mean((x - m) * (x - m), axis=0, keepdims=True)
    return jax.nn.relu((x - m) * lax.rsqrt(v + EPS))


def _node_mlp(x, Ws, bs, plain_last=False):
    n = len(Ws)
    N = x.shape[0]

    def body(*refs):
        x_r = refs[0]
        w_rs = refs[1:1 + n]
        b_rs = refs[1 + n:1 + 2 * n]
        o_r = refs[1 + 2 * n]
        h = x_r[...]
        for i in range(n):
            h = _mm(h, w_rs[i][...]) + b_rs[i][...]
            if (i < n - 1) or (not plain_last):
                h = _bn_relu(h)
        o_r[...] = h

    return pl.pallas_call(
        body, out_shape=_sds((N, H)),
    )(x, *Ws, *[b.reshape(1, H) for b in bs])


def _edgeB(xi, xj, pi16, pj16, ea, Wx, Wp, We, Wc1, bx, bp, be_, bc1, E, pd_sign):
    G = E // BE
    EP = xi.shape[0]

    def body(xi_r, xj_r, pi_r, pj_r, ea_r, wx, wp, we, wc1, bx_r, bp_r, be_r,
             bc1_r, u_o, st_o, acc):
        i = pl.program_id(0)
        xjl = (_mm(xi_r[...], wx[0:H, :]) + _mm(xj_r[...], wx[H:2 * H, :])
               + bx_r[...])
        pd = (pi_r[...][:, 0:2] - pj_r[...][:, 0:2]) * pd_sign
        pe = _mm2(pd, wp[...]) + bp_r[...]
        ee = _mm2(ea_r[...], we[...]) + be_r[...]
        u = (_mm(xjl, wc1[0:H, :]) + _mm(pe, wc1[H:2 * H, :])
             + _mm(ee, wc1[2 * H:3 * H, :]) + bc1_r[...])
        u_o[...] = u

        @pl.when(i == 0)
        def _init():
            acc[...] = jnp.zeros_like(acc)

        acc[0:1, :] += jnp.sum(u, axis=0, keepdims=True)
        acc[1:2, :] += jnp.sum(u * u, axis=0, keepdims=True)

        @pl.when(i == G - 1)
        def _fin():
            st_o[...] = acc[...]

    blk = lambda i: (i, 0)
    fix = lambda i: (0, 0)
    return pl.pallas_call(
        body,
        grid=(G,),
        in_specs=[pl.BlockSpec((BE, H), blk), pl.BlockSpec((BE, H), blk),
                  pl.BlockSpec((BE, 16), blk), pl.BlockSpec((BE, 16), blk),
                  pl.BlockSpec((BE, 2), blk),
                  pl.BlockSpec((2 * H, H), fix), pl.BlockSpec((2, H), fix),
                  pl.BlockSpec((2, H), fix), pl.BlockSpec((3 * H, H), fix),
                  pl.BlockSpec((1, H), fix), pl.BlockSpec((1, H), fix),
                  pl.BlockSpec((1, H), fix), pl.BlockSpec((1, H), fix)],
        out_specs=[pl.BlockSpec((BE, H), blk), pl.BlockSpec((2, H), fix)],
        out_shape=(_sds((EP, H)), _sds((2, H))),
        scratch_shapes=[pltpu.VMEM((2, H), F32)],
    )(xi, xj, pi16, pj16, ea, Wx, Wp, We, Wc1, bx.reshape(1, H),
      bp.reshape(1, H), be_.reshape(1, H), bc1.reshape(1, H))


def _edge2(u, st1, Wc2, bc2, E):
    G = E // BE
    EP = u.shape[0]

    def body(u_r, st_r, w2, b2, w_o, st_o, acc):
        i = pl.program_id(0)
        mu = st_r[0:1, :] / E
        var = st_r[1:2, :] / E - mu * mu
        inv = lax.rsqrt(var + EPS)
        v = jax.nn.relu((u_r[...] - mu) * inv)
        w = _mm(v, w2[...]) + b2[...]
        w_o[...] = w

        @pl.when(i == 0)
        def _init():
            acc[...] = jnp.zeros_like(acc)

        acc[0:1, :] += jnp.sum(w, axis=0, keepdims=True)
        acc[1:2, :] += jnp.sum(w * w, axis=0, keepdims=True)

        @pl.when(i == G - 1)
        def _fin():
            st_o[...] = acc[...]

    blk = lambda i: (i, 0)
    fix = lambda i: (0, 0)
    return pl.pallas_call(
        body,
        grid=(G,),
        in_specs=[pl.BlockSpec((BE, H), blk), pl.BlockSpec((2, H), fix),
                  pl.BlockSpec((H, H), fix), pl.BlockSpec((1, H), fix)],
        out_specs=[pl.BlockSpec((BE, H), blk), pl.BlockSpec((2, H), fix)],
        out_shape=(_sds((EP, H)), _sds((2, H))),
        scratch_shapes=[pltpu.VMEM((2, H), F32)],
    )(u, st1, Wc2, bc2.reshape(1, H))


def _edge3(w, st2, E):
    G = E // BE
    EP = w.shape[0]

    def body(w_r, st_r, m_o):
        mu = st_r[0:1, :] / E
        var = st_r[1:2, :] / E - mu * mu
        inv = lax.rsqrt(var + EPS)
        m_o[...] = jax.nn.relu((w_r[...] - mu) * inv)

    blk = lambda i: (i, 0)
    fix = lambda i: (0, 0)
    return pl.pallas_call(
        body,
        grid=(G,),
        in_specs=[pl.BlockSpec((BE, H), blk), pl.BlockSpec((2, H), fix)],
        out_specs=pl.BlockSpec((BE, H), blk),
        out_shape=_sds((EP, H)),
    )(w, st2)


def _addparts(part, NP):
    def body(p_r, o_r):
        o_r[...] = p_r[0:NP, :] + p_r[NP:2 * NP, :]

    return pl.pallas_call(body, out_shape=_sds((NP, H)))(part)


def _segfin(part, cnt, NP, n):
    def body(p_r, c_r, o_r):
        s = p_r[0:n, :] + p_r[NP:NP + n, :]
        o_r[...] = s / jnp.maximum(c_r[...], 1.0)

    return pl.pallas_call(body, out_shape=_sds((n, H)))(part, cnt)


def _conv(part, cnt, convW, convb, prev_cg, NP, n):
    def body(p_r, c_r, w_r, b_r, pv_r, h_o, hs_o):
        agg = (p_r[0:n, :] + p_r[NP:NP + n, :]) / jnp.maximum(c_r[...], 1.0)
        h = jax.nn.relu(_mm(agg, w_r[...]) + b_r[...])
        h_o[...] = h
        hs_o[...] = h + pv_r[...]

    return pl.pallas_call(
        body, out_shape=(_sds((n, H)), _sds((n, H))),
    )(part, cnt, convW, convb.reshape(1, H), prev_cg)


def _layerfin(part, cnt, h_og, gW, gb, W1, b1, W2, b2, prev_og, NP, n):
    def body(p_r, c_r, hog_r, gw_r, gb_r, w1_r, b1_r, w2_r, b2_r, pv_r, o_r):
        sp = (p_r[0:n, :] + p_r[NP:NP + n, :]) / jnp.maximum(c_r[...], 1.0)
        h0 = hog_r[...]
        g = jax.nn.sigmoid(_mm(h0, gw_r[0:H, :]) + _mm(sp, gw_r[H:2 * H, :])
                           + gb_r[...])
        h = g * h0 + (1.0 - g) * sp
        m = jnp.mean(h, axis=0, keepdims=True)
        v = jnp.mean((h - m) * (h - m), axis=0, keepdims=True)
        h = jax.nn.gelu((h - m) * lax.rsqrt(v + EPS))
        h = _bn_relu(_mm(h, w1_r[...]) + b1_r[...])
        h = _bn_relu(_mm(h, w2_r[...]) + b2_r[...])
        o_r[...] = h + pv_r[...]

    return pl.pallas_call(
        body, out_shape=_sds((n, H)),
    )(part, cnt, h_og, gW, gb.reshape(1, H), W1, b1.reshape(1, H),
      W2, b2.reshape(1, H), prev_og)


def _sc_mesh():
    return plsc.VectorSubcoreMesh(core_axis_name="c", subcore_axis_name="s")


def _sc_gather2(tableA, idxA, tableB, idxB):
    EP = idxA.shape[0]
    WA = tableA.shape[1]
    WB = tableB.shape[1]
    epw = EP // NW
    it = epw // CH

    @functools.partial(
        pl.kernel, mesh=_sc_mesh(),
        out_type=(_sds((EP, WA)), _sds((EP, WB))),
        scratch_types=[pltpu.VMEM((CH,), jnp.int32), pltpu.VMEM((CH, WA), F32),
                       pltpu.VMEM((CH,), jnp.int32), pltpu.VMEM((CH, WB), F32),
                       pltpu.SemaphoreType.DMA, pltpu.SemaphoreType.DMA],
    )
    def k(ta, ia, tb, ib, oa, ob, iav, rav, ibv, rbv, sa, sb):
        wid = lax.axis_index("s") * 2 + lax.axis_index("c")
        base = wid * epw

        def body(j, carry):
            off = base + j * CH
            pltpu.sync_copy(ia.at[pl.ds(off, CH)], iav)
            ca = pltpu.async_copy(ta.at[iav], rav, sa)
            pltpu.sync_copy(ib.at[pl.ds(off, CH)], ibv)
            cb = pltpu.async_copy(tb.at[ibv], rbv, sb)
            ca.wait()
            pltpu.sync_copy(rav, oa.at[pl.ds(off, CH)])
            cb.wait()
            pltpu.sync_copy(rbv, ob.at[pl.ds(off, CH)])
            return carry

        lax.fori_loop(0, it, body, 0)

    return k(tableA, idxA, tableB, idxB)


def _sc_scatter(m, idx, zeros, NP):
    EP = idx.shape[0]
    epw = EP // NW
    it = epw // CH
    rpw = NP // 16

    @functools.partial(
        pl.kernel, mesh=_sc_mesh(),
        out_type=_sds((2 * NP, H)),
        scratch_types=[pltpu.VMEM((CH,), jnp.int32), pltpu.VMEM((CH, H), F32),
                       pltpu.VMEM_SHARED((NP, H), F32)],
    )
    def k(m_h, i_h, z_h, out_h, iv, rv, acc):
        cid = lax.axis_index("c")
        sid = lax.axis_index("s")
        wid = sid * 2 + cid

        @pl.when(sid == 0)
        def _zero():
            pltpu.sync_copy(z_h, acc)

        plsc.subcore_barrier()

        def body(j, carry):
            off = wid * epw + j * CH
            pltpu.sync_copy(i_h.at[pl.ds(off, CH)], iv)
            pltpu.sync_copy(m_h.at[pl.ds(off, CH)], rv)
            pltpu.sync_copy(rv, acc.at[iv], add=True)
            return carry

        lax.fori_loop(0, it, body, 0)
        plsc.subcore_barrier()
        pltpu.sync_copy(acc.at[pl.ds(sid * rpw, rpw)],
                        out_h.at[pl.ds(cid * NP + sid * rpw, rpw)])

    return k(m, idx, zeros)


def _sc_gathscat(table, idx_s, idx_d, zeros, NP):
    EP = idx_s.shape[0]
    epw = EP // NW
    it = epw // CH
    rpw = NP // 16

    @functools.partial(
        pl.kernel, mesh=_sc_mesh(),
        out_type=_sds((2 * NP, H)),
        scratch_types=[pltpu.VMEM((CH,), jnp.int32), pltpu.VMEM((CH,), jnp.int32),
                       pltpu.VMEM((CH, H), F32), pltpu.SemaphoreType.DMA,
                       pltpu.VMEM_SHARED((NP, H), F32)],
    )
    def k(t_h, is_h, id_h, z_h, out_h, isv, idv, rv, sem, acc):
        cid = lax.axis_index("c")
        sid = lax.axis_index("s")
        wid = sid * 2 + cid

        @pl.when(sid == 0)
        def _zero():
            pltpu.sync_copy(z_h, acc)

        plsc.subcore_barrier()

        def body(j, carry):
            off = wid * epw + j * CH
            pltpu.sync_copy(is_h.at[pl.ds(off, CH)], isv)
            pltpu.async_copy(t_h.at[isv], rv, sem).wait()
            pltpu.sync_copy(id_h.at[pl.ds(off, CH)], idv)
            pltpu.sync_copy(rv, acc.at[idv], add=True)
            return carry

        lax.fori_loop(0, it, body, 0)
        plsc.subcore_barrier()
        pltpu.sync_copy(acc.at[pl.ds(sid * rpw, rpw)],
                        out_h.at[pl.ds(cid * NP + sid * rpw, rpw)])

    return k(table, idx_s, idx_d, zeros)


def _msg_op(p, x_src, x_dst, idx_src_g, idx_dst_g, idx_dst_s, pi16, pj16,
            pd_sign, eattr, zeros_dst, NP, E):
    xi, xj = _sc_gather2(x_dst, idx_dst_g, x_src, idx_src_g)
    u, st1 = _edgeB(xi, xj, pi16, pj16, eattr, p['Wx'], p['Wp'], p['We'],
                    p['Wc1'], p['bx'], p['bp'], p['be'], p['bc1'], E, pd_sign)
    w, st2 = _edge2(u, st1, p['Wc2'], p['bc2'], E)
    m = _edge3(w, st2, E)
    return _sc_scatter(m, idx_dst_s, zeros_dst, NP)


def kernel(x_og, x_cg, pos_og, pos_cg, og_to_cg_edge_index, og_to_cg_edge_attr,
           edge_index_cg, x_og_batch, x_cg_batch, params):
    N_OG, N_CG = x_og.shape[0], x_cg.shape[0]
    E1 = og_to_cg_edge_index.shape[1]
    E2 = edge_index_cg.shape[1]
    L = len(params['layers'])

    E1P = ((E1 + NW * CH - 1) // (NW * CH)) * NW * CH
    E2P = ((E2 + NW * CH - 1) // (NW * CH)) * NW * CH
    NPOG = ((N_OG + 1 + 127) // 128) * 128
    NPCG = ((N_CG + 1 + 127) // 128) * 128

    src1 = og_to_cg_edge_index[0]
    dst1 = og_to_cg_edge_index[1]
    s2 = edge_index_cg[0]
    d2 = edge_index_cg[1]

    def padg(ix, ep):
        return jnp.concatenate([ix, jnp.zeros((ep - ix.shape[0],), jnp.int32)])

    def pads(ix, ep, dummy):
        return jnp.concatenate([ix, jnp.full((ep - ix.shape[0],), dummy, jnp.int32)])

    src1g = padg(src1, E1P)
    dst1g = padg(dst1, E1P)
    src1s = pads(src1, E1P, N_OG)
    dst1s = pads(dst1, E1P, N_CG)
    s2g = padg(s2, E2P)
    d2s = pads(d2, E2P, N_CG)

    zeros_og = jnp.zeros((NPOG, H), F32)
    zeros_cg = jnp.zeros((NPCG, H), F32)
    ones1 = jnp.ones((E1P, H), F32)
    ones2 = jnp.ones((E2P, H), F32)

    cnt_dst1 = _addparts(_sc_scatter(ones1, dst1s, zeros_cg, NPCG), NPCG)[:N_CG]
    cnt_src1 = _addparts(_sc_scatter(ones1, src1s, zeros_og, NPOG), NPOG)[:N_OG]
    cnt_d2 = _addparts(_sc_scatter(ones2, d2s, zeros_cg, NPCG), NPCG)[:N_CG]

    pos_og128 = jnp.pad(pos_og, ((0, 0), (0, 126)))
    pos_cg128 = jnp.pad(pos_cg, ((0, 0), (0, 126)))
    pcg_d, pog_s = _sc_gather2(pos_cg128, dst1g, pos_og128, src1g)
    pcg_d16 = pcg_d[:, :16]
    pog_s16 = pog_s[:, :16]

    h_og = _node_mlp(x_og, params['og_proj']['Ws'], params['og_proj']['bs'])
    h_cg = _node_mlp(x_cg, params['cg_proj']['Ws'], params['cg_proj']['bs'])

    for i in range(L):
        lp = params['layers'][i]
        prev_og, prev_cg = h_og, h_cg

        part = _msg_op(lp['coars'], h_og, h_cg, src1g, dst1g, dst1s,
                       pcg_d16, pog_s16, 1.0, og_to_cg_edge_attr,
                       zeros_cg, NPCG, E1)
        h_cg = _segfin(part, cnt_dst1, NPCG, N_CG)

        part2 = _sc_gathscat(h_cg, s2g, d2s, zeros_cg, NPCG)
        h_cg, hs = _conv(part2, cnt_d2, lp['conv_W'], lp['conv_b'],
                         prev_cg, NPCG, N_CG)

        part3 = _msg_op(lp['spread'], hs, h_og, dst1g, src1g, src1s,
                        pcg_d16, pog_s16, -1.0, og_to_cg_edge_attr,
                        zeros_og, NPOG, E1)
        h_og = _layerfin(part3, cnt_src1, h_og, params['gate_W'],
                         params['gate_b'], lp['og_lin']['Ws'][0],
                         lp['og_lin']['bs'][0], lp['og_lin']['Ws'][1],
                         lp['og_lin']['bs'][1], prev_og, NPOG, N_OG)

    return _node_mlp(h_og, params['out']['Ws'], params['out']['bs'],
                     plain_last=True)


# speedup vs baseline: 1.3389x; 1.0486x over previous
import functools

import jax
import jax.numpy as jnp
from jax import lax
from jax.experimental import pallas as pl
from jax.experimental.pallas import tpu as pltpu
from jax.experimental.pallas import tpu_sc as plsc

F32 = jnp.float32
BF16 = jnp.bfloat16
H = 128
EPS = 1e-5
NW = 32
CH = 128
BE = 2000


def _sds(shape, dtype=F32):
    return jax.ShapeDtypeStruct(shape, dtype)


def _mm(a, b):
    return jnp.dot(a.astype(BF16), b.astype(BF16), preferred_element_type=F32)


def _mm2(a2, w2):
    a = a2.astype(BF16).astype(F32)
    w = w2.astype(BF16).astype(F32)
    return a[:, 0:1] * w[0:1, :] + a[:, 1:2] * w[1:2, :]


def _bn_relu(x):
    m = jnp.mean(x, axis=0, keepdims=True)
    v = jnp.mean((x - m) * (x - m), axis=0, keepdims=True)
    return jax.nn.relu((x - m) * lax.rsqrt(v + EPS))


def _node_mlp(x, Ws, bs, plain_last=False):
    n = len(Ws)
    N = x.shape[0]

    def body(*refs):
        x_r = refs[0]
        w_rs = refs[1:1 + n]
        b_rs = refs[1 + n:1 + 2 * n]
        o_r = refs[1 + 2 * n]
        h = x_r[...]
        for i in range(n):
            h = _mm(h, w_rs[i][...]) + b_rs[i][...]
            if (i < n - 1) or (not plain_last):
                h = _bn_relu(h)
        o_r[...] = h

    return pl.pallas_call(
        body, out_shape=_sds((N, H)),
    )(x, *Ws, *[b.reshape(1, H) for b in bs])


def _edgeB(xi, xj, pi16, pj16, ea, Wx, Wp, We, Wc1, bx, bp, be_, bc1, E, pd_sign):
    G = E // BE
    EP = xi.shape[0]

    def body(xi_r, xj_r, pi_r, pj_r, ea_r, wx, wp, we, wc1, bx_r, bp_r, be_r,
             bc1_r, u_o, st_o, acc):
        i = pl.program_id(0)
        xjl = (_mm(xi_r[...], wx[0:H, :]) + _mm(xj_r[...], wx[H:2 * H, :])
               + bx_r[...])
        pd = (pi_r[...][:, 0:2] - pj_r[...][:, 0:2]) * pd_sign
        pe = _mm2(pd, wp[...]) + bp_r[...]
        ee = _mm2(ea_r[...], we[...]) + be_r[...]
        u = (_mm(xjl, wc1[0:H, :]) + _mm(pe, wc1[H:2 * H, :])
             + _mm(ee, wc1[2 * H:3 * H, :]) + bc1_r[...])
        u_o[...] = u

        @pl.when(i == 0)
        def _init():
            acc[...] = jnp.zeros_like(acc)

        acc[0:1, :] += jnp.sum(u, axis=0, keepdims=True)
        acc[1:2, :] += jnp.sum(u * u, axis=0, keepdims=True)

        @pl.when(i == G - 1)
        def _fin():
            st_o[...] = acc[...]

    blk = lambda i: (i, 0)
    fix = lambda i: (0, 0)
    return pl.pallas_call(
        body,
        grid=(G,),
        in_specs=[pl.BlockSpec((BE, H), blk), pl.BlockSpec((BE, H), blk),
                  pl.BlockSpec((BE, 16), blk), pl.BlockSpec((BE, 16), blk),
                  pl.BlockSpec((BE, 2), blk),
                  pl.BlockSpec((2 * H, H), fix), pl.BlockSpec((2, H), fix),
                  pl.BlockSpec((2, H), fix), pl.BlockSpec((3 * H, H), fix),
                  pl.BlockSpec((1, H), fix), pl.BlockSpec((1, H), fix),
                  pl.BlockSpec((1, H), fix), pl.BlockSpec((1, H), fix)],
        out_specs=[pl.BlockSpec((BE, H), blk), pl.BlockSpec((2, H), fix)],
        out_shape=(_sds((EP, H)), _sds((2, H))),
        scratch_shapes=[pltpu.VMEM((2, H), F32)],
    )(xi, xj, pi16, pj16, ea, Wx, Wp, We, Wc1, bx.reshape(1, H),
      bp.reshape(1, H), be_.reshape(1, H), bc1.reshape(1, H))


def _edge2(u, st1, Wc2, bc2, E):
    G = E // BE
    EP = u.shape[0]

    def body(u_r, st_r, w2, b2, w_o, st_o, acc):
        i = pl.program_id(0)
        mu = st_r[0:1, :] / E
        var = st_r[1:2, :] / E - mu * mu
        inv = lax.rsqrt(var + EPS)
        v = jax.nn.relu((u_r[...] - mu) * inv)
        w = _mm(v, w2[...]) + b2[...]
        w_o[...] = w

        @pl.when(i == 0)
        def _init():
            acc[...] = jnp.zeros_like(acc)

        acc[0:1, :] += jnp.sum(w, axis=0, keepdims=True)
        acc[1:2, :] += jnp.sum(w * w, axis=0, keepdims=True)

        @pl.when(i == G - 1)
        def _fin():
            st_o[...] = acc[...]

    blk = lambda i: (i, 0)
    fix = lambda i: (0, 0)
    return pl.pallas_call(
        body,
        grid=(G,),
        in_specs=[pl.BlockSpec((BE, H), blk), pl.BlockSpec((2, H), fix),
                  pl.BlockSpec((H, H), fix), pl.BlockSpec((1, H), fix)],
        out_specs=[pl.BlockSpec((BE, H), blk), pl.BlockSpec((2, H), fix)],
        out_shape=(_sds((EP, H)), _sds((2, H))),
        scratch_shapes=[pltpu.VMEM((2, H), F32)],
    )(u, st1, Wc2, bc2.reshape(1, H))


def _edge3(w, st2, E):
    G = E // BE
    EP = w.shape[0]

    def body(w_r, st_r, m_o):
        mu = st_r[0:1, :] / E
        var = st_r[1:2, :] / E - mu * mu
        inv = lax.rsqrt(var + EPS)
        m_o[...] = jax.nn.relu((w_r[...] - mu) * inv)

    blk = lambda i: (i, 0)
    fix = lambda i: (0, 0)
    return pl.pallas_call(
        body,
        grid=(G,),
        in_specs=[pl.BlockSpec((BE, H), blk), pl.BlockSpec((2, H), fix)],
        out_specs=pl.BlockSpec((BE, H), blk),
        out_shape=_sds((EP, H)),
    )(w, st2)


def _addparts(part, NP):
    def body(p_r, o_r):
        o_r[...] = p_r[0:NP, :] + p_r[NP:2 * NP, :]

    return pl.pallas_call(body, out_shape=_sds((NP, H)))(part)


def _segfin(part, cnt, NP, n):
    def body(p_r, c_r, o_r):
        s = p_r[0:n, :] + p_r[NP:NP + n, :]
        o_r[...] = s / jnp.maximum(c_r[...], 1.0)

    return pl.pallas_call(body, out_shape=_sds((n, H)))(part, cnt)


def _conv(part, cnt, convW, convb, prev_cg, NP, n):
    def body(p_r, c_r, w_r, b_r, pv_r, h_o, hs_o):
        agg = (p_r[0:n, :] + p_r[NP:NP + n, :]) / jnp.maximum(c_r[...], 1.0)
        h = jax.nn.relu(_mm(agg, w_r[...]) + b_r[...])
        h_o[...] = h
        hs_o[...] = h + pv_r[...]

    return pl.pallas_call(
        body, out_shape=(_sds((n, H)), _sds((n, H))),
    )(part, cnt, convW, convb.reshape(1, H), prev_cg)


def _layerfin(part, cnt, h_og, gW, gb, W1, b1, W2, b2, prev_og, NP, n):
    def body(p_r, c_r, hog_r, gw_r, gb_r, w1_r, b1_r, w2_r, b2_r, pv_r, o_r):
        sp = (p_r[0:n, :] + p_r[NP:NP + n, :]) / jnp.maximum(c_r[...], 1.0)
        h0 = hog_r[...]
        g = jax.nn.sigmoid(_mm(h0, gw_r[0:H, :]) + _mm(sp, gw_r[H:2 * H, :])
                           + gb_r[...])
        h = g * h0 + (1.0 - g) * sp
        m = jnp.mean(h, axis=0, keepdims=True)
        v = jnp.mean((h - m) * (h - m), axis=0, keepdims=True)
        h = jax.nn.gelu((h - m) * lax.rsqrt(v + EPS))
        h = _bn_relu(_mm(h, w1_r[...]) + b1_r[...])
        h = _bn_relu(_mm(h, w2_r[...]) + b2_r[...])
        o_r[...] = h + pv_r[...]

    return pl.pallas_call(
        body, out_shape=_sds((n, H)),
    )(part, cnt, h_og, gW, gb.reshape(1, H), W1, b1.reshape(1, H),
      W2, b2.reshape(1, H), prev_og)


def _sc_mesh():
    return plsc.VectorSubcoreMesh(core_axis_name="c", subcore_axis_name="s")


def _sc_gather2(tableA, idxA, tableB, idxB):
    EP = idxA.shape[0]
    WA = tableA.shape[1]
    WB = tableB.shape[1]
    epw = EP // NW
    it = epw // CH
    K = 3
    ngrp = it // K
    rem = it - ngrp * K

    scr = ([pltpu.VMEM((CH,), jnp.int32) for _ in range(2 * K)]
           + [pltpu.VMEM((CH, WA), F32) for _ in range(K)]
           + [pltpu.VMEM((CH, WB), F32) for _ in range(K)]
           + [pltpu.SemaphoreType.DMA, pltpu.SemaphoreType.DMA,
              pltpu.SemaphoreType.DMA])

    @functools.partial(
        pl.kernel, mesh=_sc_mesh(),
        out_type=(_sds((EP, WA)), _sds((EP, WB))),
        scratch_types=scr,
    )
    def k(ta, ia, tb, ib, oa, ob, *sc):
        iva = sc[0:K]
        ivb = sc[K:2 * K]
        ra = sc[2 * K:3 * K]
        rb = sc[3 * K:4 * K]
        sem_i, sem_g, sem_s = sc[4 * K:4 * K + 3]
        wid = lax.axis_index("s") * 2 + lax.axis_index("c")
        base = wid * epw

        def do_group(j0, nk):
            cs = []
            for t in range(nk):
                off = base + (j0 + t) * CH
                cs.append(pltpu.async_copy(ia.at[pl.ds(off, CH)], iva[t], sem_i))
                cs.append(pltpu.async_copy(ib.at[pl.ds(off, CH)], ivb[t], sem_i))
            for c in cs:
                c.wait()
            cs = []
            for t in range(nk):
                cs.append(pltpu.async_copy(ta.at[iva[t]], ra[t], sem_g))
                cs.append(pltpu.async_copy(tb.at[ivb[t]], rb[t], sem_g))
            for c in cs:
                c.wait()
            cs = []
            for t in range(nk):
                off = base + (j0 + t) * CH
                cs.append(pltpu.async_copy(ra[t], oa.at[pl.ds(off, CH)], sem_s))
                cs.append(pltpu.async_copy(rb[t], ob.at[pl.ds(off, CH)], sem_s))
            for c in cs:
                c.wait()

        def body(gi, carry):
            do_group(gi * K, K)
            return carry

        lax.fori_loop(0, ngrp, body, 0)
        if rem:
            do_group(ngrp * K, rem)

    return k(tableA, idxA, tableB, idxB)


def _sc_scatter(m, idx, zeros, NP):
    EP = idx.shape[0]
    epw = EP // NW
    it = epw // CH
    K = 3
    ngrp = it // K
    rem = it - ngrp * K
    rpw = NP // 16

    scr = ([pltpu.VMEM((CH,), jnp.int32) for _ in range(K)]
           + [pltpu.VMEM((CH, H), F32) for _ in range(K)]
           + [pltpu.VMEM_SHARED((NP, H), F32), pltpu.SemaphoreType.DMA])

    @functools.partial(
        pl.kernel, mesh=_sc_mesh(),
        out_type=_sds((2 * NP, H)),
        scratch_types=scr,
    )
    def k(m_h, i_h, z_h, out_h, *sc):
        iv = sc[0:K]
        rv = sc[K:2 * K]
        acc = sc[2 * K]
        sem = sc[2 * K + 1]
        cid = lax.axis_index("c")
        sid = lax.axis_index("s")
        wid = sid * 2 + cid
        base = wid * epw

        @pl.when(sid == 0)
        def _zero():
            pltpu.sync_copy(z_h, acc)

        plsc.subcore_barrier()

        def do_group(j0, nk):
            cs = []
            for t in range(nk):
                off = base + (j0 + t) * CH
                cs.append(pltpu.async_copy(i_h.at[pl.ds(off, CH)], iv[t], sem))
                cs.append(pltpu.async_copy(m_h.at[pl.ds(off, CH)], rv[t], sem))
            for c in cs:
                c.wait()
            for t in range(nk):
                pltpu.sync_copy(rv[t], acc.at[iv[t]], add=True)

        def body(gi, carry):
            do_group(gi * K, K)
            return carry

        lax.fori_loop(0, ngrp, body, 0)
        if rem:
            do_group(ngrp * K, rem)
        plsc.subcore_barrier()
        pltpu.sync_copy(acc.at[pl.ds(sid * rpw, rpw)],
                        out_h.at[pl.ds(cid * NP + sid * rpw, rpw)])

    return k(m, idx, zeros)


def _sc_gathscat(table, idx_s, idx_d, zeros, NP):
    EP = idx_s.shape[0]
    epw = EP // NW
    it = epw // CH
    K = 2
    ngrp = it // K
    rem = it - ngrp * K
    rpw = NP // 16

    scr = ([pltpu.VMEM((CH,), jnp.int32) for _ in range(2 * K)]
           + [pltpu.VMEM((CH, H), F32) for _ in range(K)]
           + [pltpu.VMEM_SHARED((NP, H), F32),
              pltpu.SemaphoreType.DMA, pltpu.SemaphoreType.DMA])

    @functools.partial(
        pl.kernel, mesh=_sc_mesh(),
        out_type=_sds((2 * NP, H)),
        scratch_types=scr,
    )
    def k(t_h, is_h, id_h, z_h, out_h, *sc):
        isv = sc[0:K]
        idv = sc[K:2 * K]
        rv = sc[2 * K:3 * K]
        acc = sc[3 * K]
        sem_i, sem_g = sc[3 * K + 1:3 * K + 3]
        cid = lax.axis_index("c")
        sid = lax.axis_index("s")
        wid = sid * 2 + cid
        base = wid * epw

        @pl.when(sid == 0)
        def _zero():
            pltpu.sync_copy(z_h, acc)

        plsc.subcore_barrier()

        def do_group(j0, nk):
            cs = []
            for t in range(nk):
                off = base + (j0 + t) * CH
                cs.append(pltpu.async_copy(is_h.at[pl.ds(off, CH)], isv[t], sem_i))
                cs.append(pltpu.async_copy(id_h.at[pl.ds(off, CH)], idv[t], sem_i))
            for c in cs:
                c.wait()
            cs = []
            for t in range(nk):
                cs.append(pltpu.async_copy(t_h.at[isv[t]], rv[t], sem_g))
            for c in cs:
                c.wait()
            for t in range(nk):
                pltpu.sync_copy(rv[t], acc.at[idv[t]], add=True)

        def body(gi, carry):
            do_group(gi * K, K)
            return carry

        lax.fori_loop(0, ngrp, body, 0)
        if rem:
            do_group(ngrp * K, rem)
        plsc.subcore_barrier()
        pltpu.sync_copy(acc.at[pl.ds(sid * rpw, rpw)],
                        out_h.at[pl.ds(cid * NP + sid * rpw, rpw)])

    return k(table, idx_s, idx_d, zeros)


def _msg_op(p, x_src, x_dst, idx_src_g, idx_dst_g, idx_dst_s, pi16, pj16,
            pd_sign, eattr, zeros_dst, NP, E):
    xi, xj = _sc_gather2(x_dst, idx_dst_g, x_src, idx_src_g)
    u, st1 = _edgeB(xi, xj, pi16, pj16, eattr, p['Wx'], p['Wp'], p['We'],
                    p['Wc1'], p['bx'], p['bp'], p['be'], p['bc1'], E, pd_sign)
    w, st2 = _edge2(u, st1, p['Wc2'], p['bc2'], E)
    m = _edge3(w, st2, E)
    return _sc_scatter(m, idx_dst_s, zeros_dst, NP)


def kernel(x_og, x_cg, pos_og, pos_cg, og_to_cg_edge_index, og_to_cg_edge_attr,
           edge_index_cg, x_og_batch, x_cg_batch, params):
    N_OG, N_CG = x_og.shape[0], x_cg.shape[0]
    E1 = og_to_cg_edge_index.shape[1]
    E2 = edge_index_cg.shape[1]
    L = len(params['layers'])

    E1P = ((E1 + NW * CH - 1) // (NW * CH)) * NW * CH
    E2P = ((E2 + NW * CH - 1) // (NW * CH)) * NW * CH
    NPOG = ((N_OG + 1 + 127) // 128) * 128
    NPCG = ((N_CG + 1 + 127) // 128) * 128

    src1 = og_to_cg_edge_index[0]
    dst1 = og_to_cg_edge_index[1]
    s2 = edge_index_cg[0]
    d2 = edge_index_cg[1]

    def padg(ix, ep):
        return jnp.concatenate([ix, jnp.zeros((ep - ix.shape[0],), jnp.int32)])

    def pads(ix, ep, dummy):
        return jnp.concatenate([ix, jnp.full((ep - ix.shape[0],), dummy, jnp.int32)])

    src1g = padg(src1, E1P)
    dst1g = padg(dst1, E1P)
    src1s = pads(src1, E1P, N_OG)
    dst1s = pads(dst1, E1P, N_CG)
    s2g = padg(s2, E2P)
    d2s = pads(d2, E2P, N_CG)

    zeros_og = jnp.zeros((NPOG, H), F32)
    zeros_cg = jnp.zeros((NPCG, H), F32)
    ones1 = jnp.ones((E1P, H), F32)
    ones2 = jnp.ones((E2P, H), F32)

    cnt_dst1 = _addparts(_sc_scatter(ones1, dst1s, zeros_cg, NPCG), NPCG)[:N_CG]
    cnt_src1 = _addparts(_sc_scatter(ones1, src1s, zeros_og, NPOG), NPOG)[:N_OG]
    cnt_d2 = _addparts(_sc_scatter(ones2, d2s, zeros_cg, NPCG), NPCG)[:N_CG]

    pos_og128 = jnp.pad(pos_og, ((0, 0), (0, 126)))
    pos_cg128 = jnp.pad(pos_cg, ((0, 0), (0, 126)))
    pcg_d, pog_s = _sc_gather2(pos_cg128, dst1g, pos_og128, src1g)
    pcg_d16 = pcg_d[:, :16]
    pog_s16 = pog_s[:, :16]

    h_og = _node_mlp(x_og, params['og_proj']['Ws'], params['og_proj']['bs'])
    h_cg = _node_mlp(x_cg, params['cg_proj']['Ws'], params['cg_proj']['bs'])

    for i in range(L):
        lp = params['layers'][i]
        prev_og, prev_cg = h_og, h_cg

        part = _msg_op(lp['coars'], h_og, h_cg, src1g, dst1g, dst1s,
                       pcg_d16, pog_s16, 1.0, og_to_cg_edge_attr,
                       zeros_cg, NPCG, E1)
        h_cg = _segfin(part, cnt_dst1, NPCG, N_CG)

        part2 = _sc_gathscat(h_cg, s2g, d2s, zeros_cg, NPCG)
        h_cg, hs = _conv(part2, cnt_d2, lp['conv_W'], lp['conv_b'],
                         prev_cg, NPCG, N_CG)

        part3 = _msg_op(lp['spread'], hs, h_og, dst1g, src1g, src1s,
                        pcg_d16, pog_s16, -1.0, og_to_cg_edge_attr,
                        zeros_og, NPOG, E1)
        h_og = _layerfin(part3, cnt_src1, h_og, params['gate_W'],
                         params['gate_b'], lp['og_lin']['Ws'][0],
                         lp['og_lin']['bs'][0], lp['og_lin']['Ws'][1],
                         lp['og_lin']['bs'][1], prev_og, NPOG, N_OG)

    return _node_mlp(h_og, params['out']['Ws'], params['out']['bs'],
                     plain_last=True)


# speedup vs baseline: 1.4525x; 1.0848x over previous
import functools

import jax
import jax.numpy as jnp
from jax import lax
from jax.experimental import pallas as pl
from jax.experimental.pallas import tpu as pltpu
from jax.experimental.pallas import tpu_sc as plsc

F32 = jnp.float32
BF16 = jnp.bfloat16
H = 128
EPS = 1e-5
NW = 32
CH = 128
BE = 2000


def _sds(shape, dtype=F32):
    return jax.ShapeDtypeStruct(shape, dtype)


def _mm(a, b):
    return jnp.dot(a.astype(BF16), b.astype(BF16), preferred_element_type=F32)


def _mm2(a2, w2):
    a = a2.astype(BF16).astype(F32)
    w = w2.astype(BF16).astype(F32)
    return a[:, 0:1] * w[0:1, :] + a[:, 1:2] * w[1:2, :]


def _bn_relu(x):
    m = jnp.mean(x, axis=0, keepdims=True)
    v = jnp.mean((x - m) * (x - m), axis=0, keepdims=True)
    return jax.nn.relu((x - m) * lax.rsqrt(v + EPS))


def _node_mlp(x, Ws, bs, plain_last=False):
    n = len(Ws)
    N = x.shape[0]

    def body(*refs):
        x_r = refs[0]
        w_rs = refs[1:1 + n]
        b_rs = refs[1 + n:1 + 2 * n]
        o_r = refs[1 + 2 * n]
        h = x_r[...]
        for i in range(n):
            h = _mm(h, w_rs[i][...]) + b_rs[i][...]
            if (i < n - 1) or (not plain_last):
                h = _bn_relu(h)
        o_r[...] = h

    return pl.pallas_call(
        body, out_shape=_sds((N, H)),
    )(x, *Ws, *[b.reshape(1, H) for b in bs])


def _edgeB(xi, xj, pi16, pj16, ea, Wx, Wp, We, Wc1, bx, bp, be_, bc1, E, pd_sign):
    G = E // BE
    EP = xi.shape[0]

    def body(xi_r, xj_r, pi_r, pj_r, ea_r, wx, wp, we, wc1, bx_r, bp_r, be_r,
             bc1_r, u_o, st_o, acc):
        i = pl.program_id(0)
        xjl = (_mm(xi_r[...], wx[0:H, :]) + _mm(xj_r[...], wx[H:2 * H, :])
               + bx_r[...])
        pd = (pi_r[...][:, 0:2] - pj_r[...][:, 0:2]) * pd_sign
        pe = _mm2(pd, wp[...]) + bp_r[...]
        ee = _mm2(ea_r[...], we[...]) + be_r[...]
        u = (_mm(xjl, wc1[0:H, :]) + _mm(pe, wc1[H:2 * H, :])
             + _mm(ee, wc1[2 * H:3 * H, :]) + bc1_r[...])
        u_o[...] = u

        @pl.when(i == 0)
        def _init():
            acc[...] = jnp.zeros_like(acc)

        acc[0:1, :] += jnp.sum(u, axis=0, keepdims=True)
        acc[1:2, :] += jnp.sum(u * u, axis=0, keepdims=True)

        @pl.when(i == G - 1)
        def _fin():
            st_o[...] = acc[...]

    blk = lambda i: (i, 0)
    fix = lambda i: (0, 0)
    return pl.pallas_call(
        body,
        grid=(G,),
        in_specs=[pl.BlockSpec((BE, H), blk), pl.BlockSpec((BE, H), blk),
                  pl.BlockSpec((BE, 16), blk), pl.BlockSpec((BE, 16), blk),
                  pl.BlockSpec((BE, 2), blk),
                  pl.BlockSpec((2 * H, H), fix), pl.BlockSpec((2, H), fix),
                  pl.BlockSpec((2, H), fix), pl.BlockSpec((3 * H, H), fix),
                  pl.BlockSpec((1, H), fix), pl.BlockSpec((1, H), fix),
                  pl.BlockSpec((1, H), fix), pl.BlockSpec((1, H), fix)],
        out_specs=[pl.BlockSpec((BE, H), blk), pl.BlockSpec((2, H), fix)],
        out_shape=(_sds((EP, H)), _sds((2, H))),
        scratch_shapes=[pltpu.VMEM((2, H), F32)],
    )(xi, xj, pi16, pj16, ea, Wx, Wp, We, Wc1, bx.reshape(1, H),
      bp.reshape(1, H), be_.reshape(1, H), bc1.reshape(1, H))


def _edge2(u, st1, Wc2, bc2, E):
    G = E // BE
    EP = u.shape[0]

    def body(u_r, st_r, w2, b2, w_o, st_o, acc):
        i = pl.program_id(0)
        mu = st_r[0:1, :] / E
        var = st_r[1:2, :] / E - mu * mu
        inv = lax.rsqrt(var + EPS)
        v = jax.nn.relu((u_r[...] - mu) * inv)
        w = _mm(v, w2[...]) + b2[...]
        w_o[...] = w

        @pl.when(i == 0)
        def _init():
            acc[...] = jnp.zeros_like(acc)

        acc[0:1, :] += jnp.sum(w, axis=0, keepdims=True)
        acc[1:2, :] += jnp.sum(w * w, axis=0, keepdims=True)

        @pl.when(i == G - 1)
        def _fin():
            st_o[...] = acc[...]

    blk = lambda i: (i, 0)
    fix = lambda i: (0, 0)
    return pl.pallas_call(
        body,
        grid=(G,),
        in_specs=[pl.BlockSpec((BE, H), blk), pl.BlockSpec((2, H), fix),
                  pl.BlockSpec((H, H), fix), pl.BlockSpec((1, H), fix)],
        out_specs=[pl.BlockSpec((BE, H), blk), pl.BlockSpec((2, H), fix)],
        out_shape=(_sds((EP, H)), _sds((2, H))),
        scratch_shapes=[pltpu.VMEM((2, H), F32)],
    )(u, st1, Wc2, bc2.reshape(1, H))


def _edge3(w, st2, E):
    G = E // BE
    EP = w.shape[0]

    def body(w_r, st_r, m_o):
        mu = st_r[0:1, :] / E
        var = st_r[1:2, :] / E - mu * mu
        inv = lax.rsqrt(var + EPS)
        m_o[...] = jax.nn.relu((w_r[...] - mu) * inv)

    blk = lambda i: (i, 0)
    fix = lambda i: (0, 0)
    return pl.pallas_call(
        body,
        grid=(G,),
        in_specs=[pl.BlockSpec((BE, H), blk), pl.BlockSpec((2, H), fix)],
        out_specs=pl.BlockSpec((BE, H), blk),
        out_shape=_sds((EP, H)),
    )(w, st2)


def _addparts(part, NP):
    def body(p_r, o_r):
        o_r[...] = p_r[0:NP, :] + p_r[NP:2 * NP, :]

    return pl.pallas_call(body, out_shape=_sds((NP, H)))(part)


def _segfin(part, cnt, NP, n):
    def body(p_r, c_r, o_r):
        s = p_r[0:n, :] + p_r[NP:NP + n, :]
        o_r[...] = s / jnp.maximum(c_r[...], 1.0)

    return pl.pallas_call(body, out_shape=_sds((n, H)))(part, cnt)


def _conv(part, cnt, convW, convb, prev_cg, NP, n):
    def body(p_r, c_r, w_r, b_r, pv_r, h_o, hs_o):
        agg = (p_r[0:n, :] + p_r[NP:NP + n, :]) / jnp.maximum(c_r[...], 1.0)
        h = jax.nn.relu(_mm(agg, w_r[...]) + b_r[...])
        h_o[...] = h
        hs_o[...] = h + pv_r[...]

    return pl.pallas_call(
        body, out_shape=(_sds((n, H)), _sds((n, H))),
    )(part, cnt, convW, convb.reshape(1, H), prev_cg)


def _layerfin(part, cnt, h_og, gW, gb, W1, b1, W2, b2, prev_og, NP, n):
    def body(p_r, c_r, hog_r, gw_r, gb_r, w1_r, b1_r, w2_r, b2_r, pv_r, o_r):
        sp = (p_r[0:n, :] + p_r[NP:NP + n, :]) / jnp.maximum(c_r[...], 1.0)
        h0 = hog_r[...]
        g = jax.nn.sigmoid(_mm(h0, gw_r[0:H, :]) + _mm(sp, gw_r[H:2 * H, :])
                           + gb_r[...])
        h = g * h0 + (1.0 - g) * sp
        m = jnp.mean(h, axis=0, keepdims=True)
        v = jnp.mean((h - m) * (h - m), axis=0, keepdims=True)
        h = jax.nn.gelu((h - m) * lax.rsqrt(v + EPS))
        h = _bn_relu(_mm(h, w1_r[...]) + b1_r[...])
        h = _bn_relu(_mm(h, w2_r[...]) + b2_r[...])
        o_r[...] = h + pv_r[...]

    return pl.pallas_call(
        body, out_shape=_sds((n, H)),
    )(part, cnt, h_og, gW, gb.reshape(1, H), W1, b1.reshape(1, H),
      W2, b2.reshape(1, H), prev_og)


def _sc_mesh():
    return plsc.VectorSubcoreMesh(core_axis_name="c", subcore_axis_name="s")


def _sc_gather2(tableA, idxA, tableB, idxB):
    EP = idxA.shape[0]
    WA = tableA.shape[1]
    WB = tableB.shape[1]
    epw = EP // NW
    it = epw // CH
    K = 3
    ngrp = it // K
    rem = it - ngrp * K

    scr = ([pltpu.VMEM((CH,), jnp.int32) for _ in range(2 * K)]
           + [pltpu.VMEM((CH, WA), F32) for _ in range(K)]
           + [pltpu.VMEM((CH, WB), F32) for _ in range(K)]
           + [pltpu.SemaphoreType.DMA, pltpu.SemaphoreType.DMA,
              pltpu.SemaphoreType.DMA])

    @functools.partial(
        pl.kernel, mesh=_sc_mesh(),
        out_type=(_sds((EP, WA)), _sds((EP, WB))),
        scratch_types=scr,
    )
    def k(ta, ia, tb, ib, oa, ob, *sc):
        iva = sc[0:K]
        ivb = sc[K:2 * K]
        ra = sc[2 * K:3 * K]
        rb = sc[3 * K:4 * K]
        sem_i, sem_g, sem_s = sc[4 * K:4 * K + 3]
        wid = lax.axis_index("s") * 2 + lax.axis_index("c")
        base = wid * epw

        def do_group(j0, nk):
            cs = []
            for t in range(nk):
                off = base + (j0 + t) * CH
                cs.append(pltpu.async_copy(ia.at[pl.ds(off, CH)], iva[t], sem_i))
                cs.append(pltpu.async_copy(ib.at[pl.ds(off, CH)], ivb[t], sem_i))
            for c in cs:
                c.wait()
            cs = []
            for t in range(nk):
                cs.append(pltpu.async_copy(ta.at[iva[t]], ra[t], sem_g))
                cs.append(pltpu.async_copy(tb.at[ivb[t]], rb[t], sem_g))
            for c in cs:
                c.wait()
            cs = []
            for t in range(nk):
                off = base + (j0 + t) * CH
                cs.append(pltpu.async_copy(ra[t], oa.at[pl.ds(off, CH)], sem_s))
                cs.append(pltpu.async_copy(rb[t], ob.at[pl.ds(off, CH)], sem_s))
            for c in cs:
                c.wait()

        def body(gi, carry):
            do_group(gi * K, K)
            return carry

        lax.fori_loop(0, ngrp, body, 0)
        if rem:
            do_group(ngrp * K, rem)

    return k(tableA, idxA, tableB, idxB)


def _sc_gather1(tableA, idxA):
    EP = idxA.shape[0]
    WA = tableA.shape[1]
    epw = EP // NW
    it = epw // CH
    K = 3
    ngrp = it // K
    rem = it - ngrp * K

    scr = ([pltpu.VMEM((CH,), jnp.int32) for _ in range(K)]
           + [pltpu.VMEM((CH, WA), F32) for _ in range(K)]
           + [pltpu.SemaphoreType.DMA, pltpu.SemaphoreType.DMA,
              pltpu.SemaphoreType.DMA])

    @functools.partial(
        pl.kernel, mesh=_sc_mesh(),
        out_type=_sds((EP, WA)),
        scratch_types=scr,
    )
    def k(ta, ia, oa, *sc):
        iva = sc[0:K]
        ra = sc[K:2 * K]
        sem_i, sem_g, sem_s = sc[2 * K:2 * K + 3]
        wid = lax.axis_index("s") * 2 + lax.axis_index("c")
        base = wid * epw

        def do_group(j0, nk):
            cs = []
            for t in range(nk):
                off = base + (j0 + t) * CH
                cs.append(pltpu.async_copy(ia.at[pl.ds(off, CH)], iva[t], sem_i))
            for c in cs:
                c.wait()
            cs = []
            for t in range(nk):
                cs.append(pltpu.async_copy(ta.at[iva[t]], ra[t], sem_g))
            for c in cs:
                c.wait()
            cs = []
            for t in range(nk):
                off = base + (j0 + t) * CH
                cs.append(pltpu.async_copy(ra[t], oa.at[pl.ds(off, CH)], sem_s))
            for c in cs:
                c.wait()

        def body(gi, carry):
            do_group(gi * K, K)
            return carry

        lax.fori_loop(0, ngrp, body, 0)
        if rem:
            do_group(ngrp * K, rem)

    return k(tableA, idxA)


def _sc_scatter(m, idx, zeros, NP):
    EP = idx.shape[0]
    epw = EP // NW
    it = epw // CH
    K = 3
    ngrp = it // K
    rem = it - ngrp * K
    rpw = NP // 16

    scr = ([pltpu.VMEM((CH,), jnp.int32) for _ in range(K)]
           + [pltpu.VMEM((CH, H), F32) for _ in range(K)]
           + [pltpu.VMEM_SHARED((NP, H), F32), pltpu.SemaphoreType.DMA])

    @functools.partial(
        pl.kernel, mesh=_sc_mesh(),
        out_type=_sds((2 * NP, H)),
        scratch_types=scr,
    )
    def k(m_h, i_h, z_h, out_h, *sc):
        iv = sc[0:K]
        rv = sc[K:2 * K]
        acc = sc[2 * K]
        sem = sc[2 * K + 1]
        cid = lax.axis_index("c")
        sid = lax.axis_index("s")
        wid = sid * 2 + cid
        base = wid * epw

        @pl.when(sid == 0)
        def _zero():
            pltpu.sync_copy(z_h, acc)

        plsc.subcore_barrier()

        def do_group(j0, nk):
            cs = []
            for t in range(nk):
                off = base + (j0 + t) * CH
                cs.append(pltpu.async_copy(i_h.at[pl.ds(off, CH)], iv[t], sem))
                cs.append(pltpu.async_copy(m_h.at[pl.ds(off, CH)], rv[t], sem))
            for c in cs:
                c.wait()
            for t in range(nk):
                pltpu.sync_copy(rv[t], acc.at[iv[t]], add=True)

        def body(gi, carry):
            do_group(gi * K, K)
            return carry

        lax.fori_loop(0, ngrp, body, 0)
        if rem:
            do_group(ngrp * K, rem)
        plsc.subcore_barrier()
        pltpu.sync_copy(acc.at[pl.ds(sid * rpw, rpw)],
                        out_h.at[pl.ds(cid * NP + sid * rpw, rpw)])

    return k(m, idx, zeros)


def _sc_gathscat(table, idx_s, idx_d, zeros, NP):
    EP = idx_s.shape[0]
    epw = EP // NW
    it = epw // CH
    K = 2
    ngrp = it // K
    rem = it - ngrp * K
    rpw = NP // 16

    scr = ([pltpu.VMEM((CH,), jnp.int32) for _ in range(2 * K)]
           + [pltpu.VMEM((CH, H), F32) for _ in range(K)]
           + [pltpu.VMEM_SHARED((NP, H), F32),
              pltpu.SemaphoreType.DMA, pltpu.SemaphoreType.DMA])

    @functools.partial(
        pl.kernel, mesh=_sc_mesh(),
        out_type=_sds((2 * NP, H)),
        scratch_types=scr,
    )
    def k(t_h, is_h, id_h, z_h, out_h, *sc):
        isv = sc[0:K]
        idv = sc[K:2 * K]
        rv = sc[2 * K:3 * K]
        acc = sc[3 * K]
        sem_i, sem_g = sc[3 * K + 1:3 * K + 3]
        cid = lax.axis_index("c")
        sid = lax.axis_index("s")
        wid = sid * 2 + cid
        base = wid * epw

        @pl.when(sid == 0)
        def _zero():
            pltpu.sync_copy(z_h, acc)

        plsc.subcore_barrier()

        def do_group(j0, nk):
            cs = []
            for t in range(nk):
                off = base + (j0 + t) * CH
                cs.append(pltpu.async_copy(is_h.at[pl.ds(off, CH)], isv[t], sem_i))
                cs.append(pltpu.async_copy(id_h.at[pl.ds(off, CH)], idv[t], sem_i))
            for c in cs:
                c.wait()
            cs = []
            for t in range(nk):
                cs.append(pltpu.async_copy(t_h.at[isv[t]], rv[t], sem_g))
            for c in cs:
                c.wait()
            for t in range(nk):
                pltpu.sync_copy(rv[t], acc.at[idv[t]], add=True)

        def body(gi, carry):
            do_group(gi * K, K)
            return carry

        lax.fori_loop(0, ngrp, body, 0)
        if rem:
            do_group(ngrp * K, rem)
        plsc.subcore_barrier()
        pltpu.sync_copy(acc.at[pl.ds(sid * rpw, rpw)],
                        out_h.at[pl.ds(cid * NP + sid * rpw, rpw)])

    return k(table, idx_s, idx_d, zeros)


def _msg_body(p, xi, xj, idx_dst_s, pi16, pj16, pd_sign, eattr,
              zeros_dst, NP, E):
    u, st1 = _edgeB(xi, xj, pi16, pj16, eattr, p['Wx'], p['Wp'], p['We'],
                    p['Wc1'], p['bx'], p['bp'], p['be'], p['bc1'], E, pd_sign)
    w, st2 = _edge2(u, st1, p['Wc2'], p['bc2'], E)
    m = _edge3(w, st2, E)
    return _sc_scatter(m, idx_dst_s, zeros_dst, NP)


def kernel(x_og, x_cg, pos_og, pos_cg, og_to_cg_edge_index, og_to_cg_edge_attr,
           edge_index_cg, x_og_batch, x_cg_batch, params):
    N_OG, N_CG = x_og.shape[0], x_cg.shape[0]
    E1 = og_to_cg_edge_index.shape[1]
    E2 = edge_index_cg.shape[1]
    L = len(params['layers'])

    E1P = ((E1 + NW * CH - 1) // (NW * CH)) * NW * CH
    E2P = ((E2 + NW * CH - 1) // (NW * CH)) * NW * CH
    NPOG = ((N_OG + 1 + 127) // 128) * 128
    NPCG = ((N_CG + 1 + 127) // 128) * 128

    src1 = og_to_cg_edge_index[0]
    dst1 = og_to_cg_edge_index[1]
    s2 = edge_index_cg[0]
    d2 = edge_index_cg[1]

    def padg(ix, ep):
        return jnp.concatenate([ix, jnp.zeros((ep - ix.shape[0],), jnp.int32)])

    def pads(ix, ep, dummy):
        return jnp.concatenate([ix, jnp.full((ep - ix.shape[0],), dummy, jnp.int32)])

    src1g = padg(src1, E1P)
    dst1g = padg(dst1, E1P)
    src1s = pads(src1, E1P, N_OG)
    dst1s = pads(dst1, E1P, N_CG)
    s2g = padg(s2, E2P)
    d2s = pads(d2, E2P, N_CG)

    zeros_og = jnp.zeros((NPOG, H), F32)
    zeros_cg = jnp.zeros((NPCG, H), F32)
    ones1 = jnp.ones((E1P, H), F32)
    ones2 = jnp.ones((E2P, H), F32)

    cnt_dst1 = _addparts(_sc_scatter(ones1, dst1s, zeros_cg, NPCG), NPCG)[:N_CG]
    cnt_src1 = _addparts(_sc_scatter(ones1, src1s, zeros_og, NPOG), NPOG)[:N_OG]
    cnt_d2 = _addparts(_sc_scatter(ones2, d2s, zeros_cg, NPCG), NPCG)[:N_CG]

    pos_og128 = jnp.pad(pos_og, ((0, 0), (0, 126)))
    pos_cg128 = jnp.pad(pos_cg, ((0, 0), (0, 126)))
    pcg_d, pog_s = _sc_gather2(pos_cg128, dst1g, pos_og128, src1g)
    pcg_d16 = pcg_d[:, :16]
    pog_s16 = pog_s[:, :16]

    h_og = _node_mlp(x_og, params['og_proj']['Ws'], params['og_proj']['bs'])
    h_cg = _node_mlp(x_cg, params['cg_proj']['Ws'], params['cg_proj']['bs'])

    for i in range(L):
        lp = params['layers'][i]
        prev_og, prev_cg = h_og, h_cg

        xi_c, xj_c = _sc_gather2(h_cg, dst1g, h_og, src1g)
        part = _msg_body(lp['coars'], xi_c, xj_c, dst1s, pcg_d16, pog_s16,
                         1.0, og_to_cg_edge_attr, zeros_cg, NPCG, E1)
        h_cg = _segfin(part, cnt_dst1, NPCG, N_CG)

        part2 = _sc_gathscat(h_cg, s2g, d2s, zeros_cg, NPCG)
        h_cg, hs = _conv(part2, cnt_d2, lp['conv_W'], lp['conv_b'],
                         prev_cg, NPCG, N_CG)

        xj_s = _sc_gather1(hs, dst1g)
        part3 = _msg_body(lp['spread'], xj_c, xj_s, src1s, pog_s16, pcg_d16,
                          1.0, og_to_cg_edge_attr, zeros_og, NPOG, E1)
        h_og = _layerfin(part3, cnt_src1, h_og, params['gate_W'],
                         params['gate_b'], lp['og_lin']['Ws'][0],
                         lp['og_lin']['bs'][0], lp['og_lin']['Ws'][1],
                         lp['og_lin']['bs'][1], prev_og, NPOG, N_OG)

    return _node_mlp(h_og, params['out']['Ws'], params['out']['bs'],
                     plain_last=True)
